# Initial kernel scaffold; baseline (speedup 1.0000x reference)
import jax
import jax.numpy as jnp
from jax import lax
from jax.experimental import pallas as pl
from jax.experimental.pallas import tpu as pltpu
from jax.experimental.pallas import tpu_sc as plsc

_HALF = 50000
_N = 100000
_DIM = 32
_NNZ = 1600000
_BATCH = 16384
_L = 16
_CHUNK = 128
_CPG = 16
_GRP = _CHUNK * _CPG
_NGRP = 49
_EPT = _NGRP * _GRP
_PAD_NNZ = 16 * _EPT
_ACC_ROWS = 50176
_DUMP = 50000
_ZCH = 64
_ZN = (_ACC_ROWS // 16) // _ZCH
_PPW = _BATCH // 32


def _layer_body(emb, rowa, cola, vala, out, rowg, colg, valg, idxg, gath,
                zbuf, acc, semg):
    c = lax.axis_index("c")
    s = lax.axis_index("s")
    i32 = jnp.int32
    zero16 = jnp.zeros((_L,), jnp.float32)

    @pl.loop(0, _ZCH)
    def _(r):
        zbuf[r, 0:16] = zero16
        zbuf[r, 16:32] = zero16

    zbase = s * (_ACC_ROWS // 16)

    @pl.loop(0, _ZN)
    def _(j):
        pltpu.sync_copy(zbuf, acc.at[pl.ds(zbase + j * _ZCH, _ZCH)])

    plsc.subcore_barrier()

    ebase = s * _EPT

    @pl.loop(0, _NGRP)
    def _(g):
        gbase = ebase + g * _GRP
        pltpu.sync_copy(rowa.at[pl.ds(gbase, _GRP)], rowg)
        pltpu.sync_copy(cola.at[pl.ds(gbase, _GRP)], colg)
        pltpu.sync_copy(vala.at[pl.ds(gbase, _GRP)], valg)

        @pl.loop(0, _CPG)
        def _(k):
            gsl = gath.at[pl.ds(k * _CHUNK, _CHUNK)]
            pltpu.async_copy(
                emb.at[colg.at[pl.ds(k * _CHUNK, _CHUNK)]], gsl, semg
            ).wait()
            for g8 in range(8):
                e0 = k * _CHUNK + g8 * _L
                r = rowg[pl.ds(e0, _L)]
                v = valg[pl.ds(e0, _L)]
                loc = r - c * _HALF
                ok = (loc >= 0) & (loc < _HALF)
                idx = jnp.where(ok, loc, _DUMP)
                idxg[k, pl.ds(g8 * _L, _L)] = idx
                erow = e0 + lax.iota(i32, _L)
                for d in range(_DIM):
                    dcol = jnp.full((_L,), d, i32)
                    x = plsc.load_gather(gath, [erow, dcol])
                    plsc.store_scatter(gath, [erow, dcol], x * v)
            pltpu.sync_copy(gsl, acc.at[idxg.at[k]], add=True)

    plsc.subcore_barrier()

    fr = _HALF // 16
    pltpu.sync_copy(acc.at[pl.ds(s * fr, fr)],
                    out.at[pl.ds(c * _HALF + s * fr, fr)])


def _final_body(e0, e1, e2, e3, usr, itm, out, ubuf, ibuf, rows, usum, isum,
                obuf, sem):
    c = lax.axis_index("c")
    s = lax.axis_index("s")
    w = s * 2 + c
    i32 = jnp.int32
    pb = w * _PPW
    pltpu.sync_copy(usr.at[pl.ds(pb, _PPW)], ubuf)
    pltpu.sync_copy(itm.at[pl.ds(pb, _PPW)], ibuf)

    @pl.loop(0, _PPW // _L)
    def _(i):
        ibuf[pl.ds(i * _L, _L)] = ibuf[pl.ds(i * _L, _L)] + _HALF

    for idxb, accb in ((ubuf, usum), (ibuf, isum)):
        for t, tab in enumerate((e0, e1, e2, e3)):
            for k in range(_PPW // _CHUNK):
                pltpu.async_copy(
                    tab.at[idxb.at[pl.ds(k * _CHUNK, _CHUNK)]],
                    rows.at[pl.ds(k * _CHUNK, _CHUNK)], sem
                ).wait()
            if t == 0:
                @pl.loop(0, _PPW)
                def _(r):
                    accb[r, 0:16] = rows[r, 0:16]
                    accb[r, 16:32] = rows[r, 16:32]
            else:
                @pl.loop(0, _PPW)
                def _(r):
                    accb[r, 0:16] = accb[r, 0:16] + rows[r, 0:16]
                    accb[r, 16:32] = accb[r, 16:32] + rows[r, 16:32]

    scale = jnp.float32(1.0 / 16.0)

    @pl.loop(0, _PPW // _L)
    def _(g):
        pid = g * _L + lax.iota(i32, _L)
        acc = jnp.zeros((_L,), jnp.float32)
        for d in range(_DIM):
            dcol = jnp.full((_L,), d, i32)
            acc = acc + (plsc.load_gather(usum, [pid, dcol]) *
                         plsc.load_gather(isum, [pid, dcol]))
        obuf[pl.ds(g * _L, _L)] = acc * scale

    pltpu.sync_copy(obuf, out.at[pl.ds(pb, _PPW)])


_mesh = plsc.VectorSubcoreMesh(core_axis_name="c", subcore_axis_name="s")

_layer = pl.kernel(
    _layer_body,
    out_type=jax.ShapeDtypeStruct((_N, _DIM), jnp.float32),
    mesh=_mesh,
    scratch_types=[
        pltpu.VMEM((_GRP,), jnp.int32),
        pltpu.VMEM((_GRP,), jnp.int32),
        pltpu.VMEM((_GRP,), jnp.float32),
        pltpu.VMEM((_CPG, _CHUNK), jnp.int32),
        pltpu.VMEM((_GRP, _DIM), jnp.float32),
        pltpu.VMEM((_ZCH, _DIM), jnp.float32),
        pltpu.VMEM_SHARED((_ACC_ROWS, _DIM), jnp.float32),
        pltpu.SemaphoreType.DMA,
    ],
    name="lightgcn_layer",
)

_final = pl.kernel(
    _final_body,
    out_type=jax.ShapeDtypeStruct((_BATCH,), jnp.float32),
    mesh=_mesh,
    scratch_types=[
        pltpu.VMEM((_PPW,), jnp.int32),
        pltpu.VMEM((_PPW,), jnp.int32),
        pltpu.VMEM((_PPW, _DIM), jnp.float32),
        pltpu.VMEM((_PPW, _DIM), jnp.float32),
        pltpu.VMEM((_PPW, _DIM), jnp.float32),
        pltpu.VMEM((_PPW,), jnp.float32),
        pltpu.SemaphoreType.DMA,
    ],
    name="lightgcn_score",
)


def kernel(users, items, A_indices, A_values, user_emb_weight, item_emb_weight):
    row = A_indices[0].astype(jnp.int32)
    col = A_indices[1].astype(jnp.int32)
    val = A_values.astype(jnp.float32)
    npad = _PAD_NNZ - _NNZ
    rowp = jnp.concatenate([row, jnp.full((npad,), _N, jnp.int32)])
    colp = jnp.concatenate([col, jnp.zeros((npad,), jnp.int32)])
    valp = jnp.concatenate([val, jnp.zeros((npad,), jnp.float32)])
    emb0 = jnp.concatenate([user_emb_weight, item_emb_weight], axis=0)
    e1 = _layer(emb0, rowp, colp, valp)
    e2 = _layer(e1, rowp, colp, valp)
    e3 = _layer(e2, rowp, colp, valp)
    return _final(emb0, e1, e2, e3,
                  users.astype(jnp.int32), items.astype(jnp.int32))


# speedup vs baseline: 5.3575x; 5.3575x over previous
import jax
import jax.numpy as jnp
from jax import lax
from jax.experimental import pallas as pl
from jax.experimental.pallas import tpu as pltpu
from jax.experimental.pallas import tpu_sc as plsc

_HALF = 50000
_N = 100000
_HPAD = 50048
_NPAD = 2 * _HPAD
_POFF = _HPAD - _HALF
_DIM = 32
_NNZ = 1600000
_BATCH = 16384
_L = 16
_CHUNK = 128
_CPG = 4
_GRP = _CHUNK * _CPG
_NGRP = 196
_EPT = _NGRP * _GRP
_PAD_NNZ = 16 * _EPT
_ACC_ROWS = 50176
_DUMP = 50000
_ZCH = 64
_ZN = (_ACC_ROWS // 16) // _ZCH
_PPW = _BATCH // 32


def _layer_body(emb, rowa, cola, vala, out, rowg, colg, valg, idxg, gath,
                zbuf, acc, semg):
    c = lax.axis_index("c")
    s = lax.axis_index("s")
    i32 = jnp.int32
    zero16 = jnp.zeros((_L,), jnp.float32)

    @pl.loop(0, _ZCH)
    def _(r):
        zbuf[r, 0:16] = zero16
        zbuf[r, 16:32] = zero16

    zbase = s * (_ACC_ROWS // 16)

    @pl.loop(0, _ZN)
    def _(j):
        pltpu.sync_copy(zbuf, acc.at[pl.ds(zbase + j * _ZCH, _ZCH)])

    plsc.subcore_barrier()

    ebase = s * _EPT

    @pl.loop(0, _NGRP)
    def _(g):
        gbase = ebase + g * _GRP
        pltpu.sync_copy(rowa.at[pl.ds(gbase, _GRP)], rowg)
        pltpu.sync_copy(cola.at[pl.ds(gbase, _GRP)], colg)
        pltpu.sync_copy(vala.at[pl.ds(gbase, _GRP)], valg)

        @pl.loop(0, _GRP // _L)
        def _(i):
            x = colg[pl.ds(i * _L, _L)]
            colg[pl.ds(i * _L, _L)] = jnp.where(x >= _HALF, x + _POFF, x)

        @pl.loop(0, _CPG)
        def _(k):
            gsl = gath.at[pl.ds(k * _CHUNK, _CHUNK)]
            pltpu.async_copy(
                emb.at[colg.at[pl.ds(k * _CHUNK, _CHUNK)]], gsl, semg
            ).wait()
            for g8 in range(8):
                e0 = k * _CHUNK + g8 * _L
                r = rowg[pl.ds(e0, _L)]
                v16 = valg[pl.ds(e0, _L)]
                loc = r - c * _HALF
                ok = (loc >= 0) & (loc < _HALF)
                idx = jnp.where(ok, loc, _DUMP)
                idxg[k, pl.ds(g8 * _L, _L)] = idx
                for j in range(_L):
                    e = e0 + j
                    vs = v16[j]
                    gath[e, 0:16] = gath[e, 0:16] * vs
                    gath[e, 16:32] = gath[e, 16:32] * vs
            pltpu.sync_copy(gsl, acc.at[idxg.at[k]], add=True)

    plsc.subcore_barrier()

    fr = _HPAD // 16
    pltpu.sync_copy(acc.at[pl.ds(s * fr, fr)],
                    out.at[pl.ds(c * _HPAD + s * fr, fr)])


def _final_body(e0, e1, e2, e3, usr, itm, uout, iout, ubuf, ibuf, rows, usum,
                isum, sem):
    c = lax.axis_index("c")
    s = lax.axis_index("s")
    w = s * 2 + c
    pb = w * _PPW
    pltpu.sync_copy(usr.at[pl.ds(pb, _PPW)], ubuf)
    pltpu.sync_copy(itm.at[pl.ds(pb, _PPW)], ibuf)

    @pl.loop(0, _PPW // _L)
    def _(i):
        ibuf[pl.ds(i * _L, _L)] = ibuf[pl.ds(i * _L, _L)] + _HPAD

    for idxb, accb, dst in ((ubuf, usum, uout), (ibuf, isum, iout)):
        for t, tab in enumerate((e0, e1, e2, e3)):
            for k in range(_PPW // _CHUNK):
                pltpu.async_copy(
                    tab.at[idxb.at[pl.ds(k * _CHUNK, _CHUNK)]],
                    rows.at[pl.ds(k * _CHUNK, _CHUNK)], sem
                ).wait()
            if t == 0:
                @pl.loop(0, _PPW)
                def _(r):
                    accb[r, 0:16] = rows[r, 0:16]
                    accb[r, 16:32] = rows[r, 16:32]
            else:
                @pl.loop(0, _PPW)
                def _(r):
                    accb[r, 0:16] = accb[r, 0:16] + rows[r, 0:16]
                    accb[r, 16:32] = accb[r, 16:32] + rows[r, 16:32]
        pltpu.sync_copy(accb, dst.at[pl.ds(pb, _PPW)])


def _dot_body(u_ref, i_ref, o_ref):
    o_ref[...] = jnp.sum(u_ref[...] * i_ref[...], axis=1) * (1.0 / 16.0)


_mesh = plsc.VectorSubcoreMesh(core_axis_name="c", subcore_axis_name="s")

_layer = pl.kernel(
    _layer_body,
    out_type=jax.ShapeDtypeStruct((_NPAD, _DIM), jnp.float32),
    mesh=_mesh,
    scratch_types=[
        pltpu.VMEM((_GRP,), jnp.int32),
        pltpu.VMEM((_GRP,), jnp.int32),
        pltpu.VMEM((_GRP,), jnp.float32),
        pltpu.VMEM((_CPG, _CHUNK), jnp.int32),
        pltpu.VMEM((_GRP, _DIM), jnp.float32),
        pltpu.VMEM((_ZCH, _DIM), jnp.float32),
        pltpu.VMEM_SHARED((_ACC_ROWS, _DIM), jnp.float32),
        pltpu.SemaphoreType.DMA,
    ],
    name="lightgcn_layer",
    compiler_params=pltpu.CompilerParams(use_tc_tiling_on_sc=False),
)

_final = pl.kernel(
    _final_body,
    out_type=(jax.ShapeDtypeStruct((_BATCH, _DIM), jnp.float32),
              jax.ShapeDtypeStruct((_BATCH, _DIM), jnp.float32)),
    mesh=_mesh,
    scratch_types=[
        pltpu.VMEM((_PPW,), jnp.int32),
        pltpu.VMEM((_PPW,), jnp.int32),
        pltpu.VMEM((_PPW, _DIM), jnp.float32),
        pltpu.VMEM((_PPW, _DIM), jnp.float32),
        pltpu.VMEM((_PPW, _DIM), jnp.float32),
        pltpu.SemaphoreType.DMA,
    ],
    name="lightgcn_gather",
    compiler_params=pltpu.CompilerParams(use_tc_tiling_on_sc=False),
)

_dot = pl.pallas_call(
    _dot_body,
    out_shape=jax.ShapeDtypeStruct((_BATCH,), jnp.float32),
    name="lightgcn_dot",
)


def kernel(users, items, A_indices, A_values, user_emb_weight, item_emb_weight):
    row = A_indices[0].astype(jnp.int32)
    col = A_indices[1].astype(jnp.int32)
    val = A_values.astype(jnp.float32)
    npad = _PAD_NNZ - _NNZ
    rowp = jnp.concatenate([row, jnp.full((npad,), _N, jnp.int32)])
    colp = jnp.concatenate([col, jnp.zeros((npad,), jnp.int32)])
    valp = jnp.concatenate([val, jnp.zeros((npad,), jnp.float32)])
    zpad = jnp.zeros((_POFF, _DIM), jnp.float32)
    emb0 = jnp.concatenate([user_emb_weight, zpad, item_emb_weight, zpad],
                           axis=0)
    e1 = _layer(emb0, rowp, colp, valp)
    e2 = _layer(e1, rowp, colp, valp)
    e3 = _layer(e2, rowp, colp, valp)
    ug, ig = _final(emb0, e1, e2, e3,
                    users.astype(jnp.int32), items.astype(jnp.int32))
    return _dot(ug, ig)


# speedup vs baseline: 6.4664x; 1.2070x over previous
import jax
import jax.numpy as jnp
from jax import lax
from jax.experimental import pallas as pl
from jax.experimental.pallas import tpu as pltpu
from jax.experimental.pallas import tpu_sc as plsc

_HALF = 50000
_N = 100000
_HPAD = 50048
_NPAD = 2 * _HPAD
_POFF = _HPAD - _HALF
_DIM = 32
_NNZ = 1600000
_BATCH = 16384
_L = 16
_CHUNK = 128
_CPG = 3
_GRP = _CHUNK * _CPG
_NGRP = 262
_EPT = _NGRP * _GRP
_PAD_NNZ = 16 * _EPT
_PAD_ALLOC = _PAD_NNZ + _GRP
_ACC_ROWS = 50176
_DUMP = 50000
_ZCH = 64
_ZN = (_ACC_ROWS // 16) // _ZCH
_PPW = _BATCH // 32


def _layer_body(emb, rowa, cola, vala, out,
                rowgA, colgA, valgA, idxgA, gathA,
                rowgB, colgB, valgB, idxgB, gathB,
                zbuf, acc, semlA, semlB, semgA, semgB, semsA, semsB, semz):
    c = lax.axis_index("c")
    s = lax.axis_index("s")
    zero16 = jnp.zeros((_L,), jnp.float32)

    @pl.loop(0, _ZCH)
    def _(r):
        zbuf[r, 0:16] = zero16
        zbuf[r, 16:32] = zero16

    zbase = s * (_ACC_ROWS // 16)

    @pl.loop(0, _ZN)
    def _(j):
        pltpu.async_copy(zbuf, acc.at[pl.ds(zbase + j * _ZCH, _ZCH)], semz)

    @pl.loop(0, _ZN)
    def _(j):
        pltpu.make_async_copy(
            zbuf, acc.at[pl.ds(zbase + j * _ZCH, _ZCH)], semz).wait()

    plsc.subcore_barrier()

    ebase = s * _EPT

    def stage(g, rowg, colg, valg, seml):
        gb = ebase + g * _GRP
        pltpu.async_copy(rowa.at[pl.ds(gb, _GRP)], rowg, seml)
        pltpu.async_copy(cola.at[pl.ds(gb, _GRP)], colg, seml)
        pltpu.async_copy(vala.at[pl.ds(gb, _GRP)], valg, seml)

    def wstage(rowg, colg, valg, seml):
        pltpu.make_async_copy(rowa.at[pl.ds(0, _GRP)], rowg, seml).wait()
        pltpu.make_async_copy(cola.at[pl.ds(0, _GRP)], colg, seml).wait()
        pltpu.make_async_copy(vala.at[pl.ds(0, _GRP)], valg, seml).wait()

    def drain_scat(gath, sems):
        for k in range(_CPG):
            pltpu.make_async_copy(
                emb.at[pl.ds(0, _CHUNK)],
                gath.at[pl.ds(k * _CHUNK, _CHUNK)], sems).wait()

    def part(g, bufp, bufq, next_g, drain):
        rowg, colg, valg, idxg, gath, seml, semg, sems = bufp
        rowq, colq, valq, _, _, semlq, _, _ = bufq
        wstage(rowg, colg, valg, seml)

        @pl.loop(0, _GRP // _L)
        def _(i):
            x = colg[pl.ds(i * _L, _L)]
            colg[pl.ds(i * _L, _L)] = jnp.where(x >= _HALF, x + _POFF, x)

        if drain:
            drain_scat(gath, sems)
        for k in range(_CPG):
            pltpu.async_copy(emb.at[colg.at[pl.ds(k * _CHUNK, _CHUNK)]],
                             gath.at[pl.ds(k * _CHUNK, _CHUNK)], semg)
        stage(next_g, rowq, colq, valq, semlq)

        @pl.loop(0, _CPG)
        def _(k):
            pltpu.make_async_copy(
                emb.at[pl.ds(0, _CHUNK)],
                gath.at[pl.ds(k * _CHUNK, _CHUNK)], semg).wait()
            for g8 in range(_CHUNK // _L):
                e0 = k * _CHUNK + g8 * _L
                r = rowg[pl.ds(e0, _L)]
                v16 = valg[pl.ds(e0, _L)]
                loc = r - c * _HALF
                ok = (loc >= 0) & (loc < _HALF)
                idx = jnp.where(ok, loc, _DUMP)
                idxg[k, pl.ds(g8 * _L, _L)] = idx
                for j in range(_L):
                    e = e0 + j
                    vs = v16[j]
                    gath[e, 0:16] = gath[e, 0:16] * vs
                    gath[e, 16:32] = gath[e, 16:32] * vs
            pltpu.async_copy(gath.at[pl.ds(k * _CHUNK, _CHUNK)],
                             acc.at[idxg.at[k]], sems, add=True)

    bufA = (rowgA, colgA, valgA, idxgA, gathA, semlA, semgA, semsA)
    bufB = (rowgB, colgB, valgB, idxgB, gathB, semlB, semgB, semsB)

    stage(0, rowgA, colgA, valgA, semlA)
    part(0, bufA, bufB, 1, drain=False)
    part(1, bufB, bufA, 2, drain=False)

    @pl.loop(1, _NGRP // 2)
    def _(t):
        part(2 * t, bufA, bufB, 2 * t + 1, drain=True)
        part(2 * t + 1, bufB, bufA, 2 * t + 2, drain=True)

    wstage(rowgA, colgA, valgA, semlA)
    drain_scat(gathA, semsA)
    drain_scat(gathB, semsB)

    plsc.subcore_barrier()

    fr = _HPAD // 16
    pltpu.sync_copy(acc.at[pl.ds(s * fr, fr)],
                    out.at[pl.ds(c * _HPAD + s * fr, fr)])


def _final_body(e0, e1, e2, e3, usr, itm, uout, iout, ubuf, ibuf, rows, usum,
                isum, sem):
    c = lax.axis_index("c")
    s = lax.axis_index("s")
    w = s * 2 + c
    pb = w * _PPW
    pltpu.sync_copy(usr.at[pl.ds(pb, _PPW)], ubuf)
    pltpu.sync_copy(itm.at[pl.ds(pb, _PPW)], ibuf)

    @pl.loop(0, _PPW // _L)
    def _(i):
        ibuf[pl.ds(i * _L, _L)] = ibuf[pl.ds(i * _L, _L)] + _HPAD

    for idxb, accb, dst in ((ubuf, usum, uout), (ibuf, isum, iout)):
        for t, tab in enumerate((e0, e1, e2, e3)):
            for k in range(_PPW // _CHUNK):
                pltpu.async_copy(
                    tab.at[idxb.at[pl.ds(k * _CHUNK, _CHUNK)]],
                    rows.at[pl.ds(k * _CHUNK, _CHUNK)], sem
                ).wait()
            if t == 0:
                @pl.loop(0, _PPW)
                def _(r):
                    accb[r, 0:16] = rows[r, 0:16]
                    accb[r, 16:32] = rows[r, 16:32]
            else:
                @pl.loop(0, _PPW)
                def _(r):
                    accb[r, 0:16] = accb[r, 0:16] + rows[r, 0:16]
                    accb[r, 16:32] = accb[r, 16:32] + rows[r, 16:32]
        pltpu.sync_copy(accb, dst.at[pl.ds(pb, _PPW)])


def _dot_body(u_ref, i_ref, o_ref):
    o_ref[...] = jnp.sum(u_ref[...] * i_ref[...], axis=1) * (1.0 / 16.0)


_mesh = plsc.VectorSubcoreMesh(core_axis_name="c", subcore_axis_name="s")

_layer = pl.kernel(
    _layer_body,
    out_type=jax.ShapeDtypeStruct((_NPAD, _DIM), jnp.float32),
    mesh=_mesh,
    scratch_types=[
        pltpu.VMEM((_GRP,), jnp.int32),
        pltpu.VMEM((_GRP,), jnp.int32),
        pltpu.VMEM((_GRP,), jnp.float32),
        pltpu.VMEM((_CPG, _CHUNK), jnp.int32),
        pltpu.VMEM((_GRP, _DIM), jnp.float32),
        pltpu.VMEM((_GRP,), jnp.int32),
        pltpu.VMEM((_GRP,), jnp.int32),
        pltpu.VMEM((_GRP,), jnp.float32),
        pltpu.VMEM((_CPG, _CHUNK), jnp.int32),
        pltpu.VMEM((_GRP, _DIM), jnp.float32),
        pltpu.VMEM((_ZCH, _DIM), jnp.float32),
        pltpu.VMEM_SHARED((_ACC_ROWS, _DIM), jnp.float32),
        pltpu.SemaphoreType.DMA,
        pltpu.SemaphoreType.DMA,
        pltpu.SemaphoreType.DMA,
        pltpu.SemaphoreType.DMA,
        pltpu.SemaphoreType.DMA,
        pltpu.SemaphoreType.DMA,
        pltpu.SemaphoreType.DMA,
    ],
    name="lightgcn_layer",
    compiler_params=pltpu.CompilerParams(use_tc_tiling_on_sc=False),
)

_final = pl.kernel(
    _final_body,
    out_type=(jax.ShapeDtypeStruct((_BATCH, _DIM), jnp.float32),
              jax.ShapeDtypeStruct((_BATCH, _DIM), jnp.float32)),
    mesh=_mesh,
    scratch_types=[
        pltpu.VMEM((_PPW,), jnp.int32),
        pltpu.VMEM((_PPW,), jnp.int32),
        pltpu.VMEM((_PPW, _DIM), jnp.float32),
        pltpu.VMEM((_PPW, _DIM), jnp.float32),
        pltpu.VMEM((_PPW, _DIM), jnp.float32),
        pltpu.SemaphoreType.DMA,
    ],
    name="lightgcn_gather",
    compiler_params=pltpu.CompilerParams(use_tc_tiling_on_sc=False),
)

_dot = pl.pallas_call(
    _dot_body,
    out_shape=jax.ShapeDtypeStruct((_BATCH,), jnp.float32),
    name="lightgcn_dot",
)


def kernel(users, items, A_indices, A_values, user_emb_weight, item_emb_weight):
    row = A_indices[0].astype(jnp.int32)
    col = A_indices[1].astype(jnp.int32)
    val = A_values.astype(jnp.float32)
    npad = _PAD_ALLOC - _NNZ
    rowp = jnp.concatenate([row, jnp.full((npad,), _N, jnp.int32)])
    colp = jnp.concatenate([col, jnp.zeros((npad,), jnp.int32)])
    valp = jnp.concatenate([val, jnp.zeros((npad,), jnp.float32)])
    zpad = jnp.zeros((_POFF, _DIM), jnp.float32)
    emb0 = jnp.concatenate([user_emb_weight, zpad, item_emb_weight, zpad],
                           axis=0)
    e1 = _layer(emb0, rowp, colp, valp)
    e2 = _layer(e1, rowp, colp, valp)
    e3 = _layer(e2, rowp, colp, valp)
    ug, ig = _final(emb0, e1, e2, e3,
                    users.astype(jnp.int32), items.astype(jnp.int32))
    return _dot(ug, ig)
